# baseline (speedup 1.0000x reference)
import jax
import jax.numpy as jnp
from jax.experimental import pallas as pl


def _gat_conv(x, edge_index, W, a_src, a_dst, b, heads, out_c, concat):
    N = x.shape[0]
    h = (x @ W).reshape(N, heads, out_c)
    src = edge_index[0]
    dst = edge_index[1]
    alpha_src = (h * a_src).sum(-1)
    alpha_dst = (h * a_dst).sum(-1)
    e = alpha_src[src] + alpha_dst[dst]
    e = jax.nn.leaky_relu(e, negative_slope=0.2)
    emax = jax.ops.segment_max(e, dst, num_segments=N)
    emax = jnp.where(jnp.isfinite(emax), emax, 0.0)
    ex = jnp.exp(e - emax[dst])
    esum = jax.ops.segment_sum(ex, dst, num_segments=N)
    alpha = ex / (esum[dst] + 1e-16)
    msg = h[src] * alpha[:, :, None]
    out = jax.ops.segment_sum(msg, dst, num_segments=N)
    if concat:
        out = out.reshape(N, heads * out_c)
    else:
        out = out.mean(axis=1)
    return out + b


def _matmul_kernel(x_ref, w_ref, b_ref, o_ref):
    o_ref[...] = x_ref[...] @ w_ref[...] + b_ref[...]


def kernel(x, edge_index, W1, a_src1, a_dst1, b1, W2, a_src2, a_dst2, b2,
           W3, a_src3, a_dst3, b3, W_out, b_out):
    h = jax.nn.elu(_gat_conv(x, edge_index, W1, a_src1, a_dst1, b1, 8, 32, True))
    h = jax.nn.elu(_gat_conv(h, edge_index, W2, a_src2, a_dst2, b2, 8, 16, True))
    h = jax.nn.elu(_gat_conv(h, edge_index, W3, a_src3, a_dst3, b3, 1, 64, False))
    N = h.shape[0]
    out = pl.pallas_call(
        _matmul_kernel,
        out_shape=jax.ShapeDtypeStruct((N, W_out.shape[1]), jnp.float32),
    )(h, W_out, b_out[None, :])
    return out


# speedup vs baseline: 18.5491x; 18.5491x over previous
import functools

import jax
import jax.numpy as jnp
from jax import lax
from jax.experimental import pallas as pl
from jax.experimental.pallas import tpu as pltpu
from jax.experimental.pallas import tpu_sc as plsc

N = 10000
E = 320000
NC = 2
NS = 16
EPW = E // NS
CH = 80
NCHUNK = EPW // CH
RPW = 624
RTAIL = N - NS * RPW
HP = 16

_f32 = jnp.float32


def _make_sc_edge_kernel(H, C, F2):
    HH = max(H // 2, 1)
    VJ = (F2 // 16) // HH

    mesh = plsc.VectorSubcoreMesh(core_axis_name="c", subcore_axis_name="s")

    def body(s_hbm, d_hbm, hlo_hbm, hhi_hbm, src_hbm, dst_hbm, m_hbm,
             zagg_hbm, zes_hbm,
             agg_out, es_out,
             sidx, didx, srows, drows, exch, hrows, msg, mv,
             agg_sh, es_sh, sem, gsem):
        c = lax.axis_index("c")
        s = lax.axis_index("s")
        r0 = s * RPW
        pltpu.sync_copy(zagg_hbm.at[pl.ds(r0, RPW)], agg_sh.at[pl.ds(r0, RPW)])
        pltpu.sync_copy(zes_hbm.at[pl.ds(r0, RPW)], es_sh.at[pl.ds(r0, RPW)])

        @pl.when(s == NS - 1)
        def _():
            pltpu.sync_copy(zagg_hbm.at[pl.ds(NS * RPW, RTAIL)],
                            agg_sh.at[pl.ds(NS * RPW, RTAIL)])
            pltpu.sync_copy(zes_hbm.at[pl.ds(NS * RPW, RTAIL)],
                            es_sh.at[pl.ds(NS * RPW, RTAIL)])

        pltpu.sync_copy(m_hbm, mv)
        plsc.subcore_barrier()

        ebase = s * EPW

        def chunk_body(k, carry):
            b = ebase + k * CH
            pltpu.sync_copy(src_hbm.at[pl.ds(b, CH)], sidx)
            pltpu.sync_copy(dst_hbm.at[pl.ds(b, CH)], didx)
            pltpu.async_copy(s_hbm.at[sidx], srows, sem).wait()
            pltpu.async_copy(d_hbm.at[didx], drows, sem).wait()

            @pl.when(c == 0)
            def _():
                pltpu.async_copy(hlo_hbm.at[sidx], hrows, gsem).wait()

            @pl.when(c == 1)
            def _():
                pltpu.async_copy(hhi_hbm.at[sidx], hrows, gsem).wait()

            m = mv[...]

            def edge_body(e, carry2):
                v = srows[e] + drows[e]
                v = jnp.where(v > 0.0, v, 0.2 * v)
                ev = jnp.exp(v - m)
                exch[e] = ev
                for h2 in range(HH):
                    if H > 1:
                        hd = c * HH + h2
                    else:
                        hd = h2
                    idx = jnp.full((16,), hd, dtype=jnp.int32)
                    mlt = jnp.take_along_axis(
                        ev, idx, axis=0,
                        mode=lax.GatherScatterMode.PROMISE_IN_BOUNDS)
                    for jj in range(VJ):
                        j = h2 * VJ + jj
                        msg[e, pl.ds(j * 16, 16)] = (
                            hrows[e, pl.ds(j * 16, 16)] * mlt)
                return carry2

            lax.fori_loop(0, CH, edge_body, 0)
            pltpu.sync_copy(exch, es_sh.at[didx], add=True)
            pltpu.sync_copy(msg, agg_sh.at[didx], add=True)
            return carry

        lax.fori_loop(0, NCHUNK, chunk_body, 0)
        plsc.subcore_barrier()
        pltpu.sync_copy(agg_sh.at[pl.ds(r0, RPW)],
                        agg_out.at[c, pl.ds(r0, RPW)])
        pltpu.sync_copy(es_sh.at[pl.ds(r0, RPW)],
                        es_out.at[c, pl.ds(r0, RPW)])

        @pl.when(s == NS - 1)
        def _():
            pltpu.sync_copy(agg_sh.at[pl.ds(NS * RPW, RTAIL)],
                            agg_out.at[c, pl.ds(NS * RPW, RTAIL)])
            pltpu.sync_copy(es_sh.at[pl.ds(NS * RPW, RTAIL)],
                            es_out.at[c, pl.ds(NS * RPW, RTAIL)])

    return pl.kernel(
        body,
        out_type=(jax.ShapeDtypeStruct((NC, N, F2), _f32),
                  jax.ShapeDtypeStruct((NC, N, HP), _f32)),
        mesh=mesh,
        compiler_params=pltpu.CompilerParams(use_tc_tiling_on_sc=False),
        scratch_types=[
            pltpu.VMEM((CH,), jnp.int32),
            pltpu.VMEM((CH,), jnp.int32),
            pltpu.VMEM((CH, HP), _f32),
            pltpu.VMEM((CH, HP), _f32),
            pltpu.VMEM((CH, HP), _f32),
            pltpu.VMEM((CH, F2), _f32),
            pltpu.VMEM((CH, F2), _f32),
            pltpu.VMEM((16,), _f32),
            pltpu.VMEM_SHARED((N, F2), _f32),
            pltpu.VMEM_SHARED((N, HP), _f32),
            pltpu.SemaphoreType.DMA,
            pltpu.SemaphoreType.DMA,
        ],
    )


BN = 1000


def _dense_tail(h, asrc_ref, adst_ref, i, hlo_ref, hhi_ref, s_ref, d_ref,
                ms_ref, md_ref):
    F2 = h.shape[1] // 2
    hlo_ref[...] = h[:, :F2]
    hhi_ref[...] = h[:, F2:]
    sblk = jnp.dot(h, asrc_ref[...], preferred_element_type=_f32)
    dblk = jnp.dot(h, adst_ref[...], preferred_element_type=_f32)
    s_ref[...] = sblk
    d_ref[...] = dblk
    cms = jnp.broadcast_to(jnp.max(sblk, axis=0, keepdims=True), (8, HP))
    cmd = jnp.broadcast_to(jnp.max(dblk, axis=0, keepdims=True), (8, HP))

    @pl.when(i == 0)
    def _():
        ms_ref[...] = cms
        md_ref[...] = cmd

    @pl.when(i != 0)
    def _():
        ms_ref[...] = jnp.maximum(ms_ref[...], cms)
        md_ref[...] = jnp.maximum(md_ref[...], cmd)


def _producer_body(y_ref, w_ref, asrc_ref, adst_ref,
                   hlo_ref, hhi_ref, s_ref, d_ref, ms_ref, md_ref):
    i = pl.program_id(0)
    h = jnp.dot(y_ref[...], w_ref[...], preferred_element_type=_f32)
    _dense_tail(h, asrc_ref, adst_ref, i, hlo_ref, hhi_ref, s_ref, d_ref,
                ms_ref, md_ref)


def _elu(y):
    return jnp.where(y > 0.0, y, jnp.exp(jnp.minimum(y, 0.0)) - 1.0)


def _combine(agg_ref, es_ref, bias_ref, ssel_ref):
    ycat = jnp.concatenate([agg_ref[0], agg_ref[1]], axis=1)
    inv = 1.0 / (es_ref[0] + 1e-16)
    rep = jnp.dot(inv, ssel_ref[...], preferred_element_type=_f32)
    return _elu(ycat * rep + bias_ref[...])


def _mid_body(agg_ref, es_ref, bias_ref, ssel_ref, w_ref, asrc_ref, adst_ref,
              hlo_ref, hhi_ref, s_ref, d_ref, ms_ref, md_ref):
    i = pl.program_id(0)
    y = _combine(agg_ref, es_ref, bias_ref, ssel_ref)
    h = jnp.dot(y, w_ref[...], preferred_element_type=_f32)
    _dense_tail(h, asrc_ref, adst_ref, i, hlo_ref, hhi_ref, s_ref, d_ref,
                ms_ref, md_ref)


def _final_body(agg_ref, es_ref, bias_ref, ssel_ref, wout_ref, bout_ref,
                o_ref):
    y = _combine(agg_ref, es_ref, bias_ref, ssel_ref)
    o_ref[...] = jnp.dot(y, wout_ref[...],
                         preferred_element_type=_f32) + bout_ref[...]


def _producer_call(y, W, Asrc, Adst):
    Fin = y.shape[1]
    F = W.shape[1]
    F2 = F // 2
    return pl.pallas_call(
        _producer_body,
        grid=(N // BN,),
        in_specs=[
            pl.BlockSpec((BN, Fin), lambda i: (i, 0)),
            pl.BlockSpec((Fin, F), lambda i: (0, 0)),
            pl.BlockSpec((F, HP), lambda i: (0, 0)),
            pl.BlockSpec((F, HP), lambda i: (0, 0)),
        ],
        out_specs=[
            pl.BlockSpec((BN, F2), lambda i: (i, 0)),
            pl.BlockSpec((BN, F2), lambda i: (i, 0)),
            pl.BlockSpec((BN, HP), lambda i: (i, 0)),
            pl.BlockSpec((BN, HP), lambda i: (i, 0)),
            pl.BlockSpec((8, HP), lambda i: (0, 0)),
            pl.BlockSpec((8, HP), lambda i: (0, 0)),
        ],
        out_shape=[
            jax.ShapeDtypeStruct((N, F2), _f32),
            jax.ShapeDtypeStruct((N, F2), _f32),
            jax.ShapeDtypeStruct((N, HP), _f32),
            jax.ShapeDtypeStruct((N, HP), _f32),
            jax.ShapeDtypeStruct((8, HP), _f32),
            jax.ShapeDtypeStruct((8, HP), _f32),
        ],
    )(y, W, Asrc, Adst)


def _mid_call(agg, es, bias, ssel, W, Asrc, Adst):
    F2p = agg.shape[2]
    Fp = 2 * F2p
    F = W.shape[1]
    F2 = F // 2
    return pl.pallas_call(
        _mid_body,
        grid=(N // BN,),
        in_specs=[
            pl.BlockSpec((2, BN, F2p), lambda i: (0, i, 0)),
            pl.BlockSpec((1, BN, HP), lambda i: (0, i, 0)),
            pl.BlockSpec((1, Fp), lambda i: (0, 0)),
            pl.BlockSpec((HP, Fp), lambda i: (0, 0)),
            pl.BlockSpec((Fp, F), lambda i: (0, 0)),
            pl.BlockSpec((F, HP), lambda i: (0, 0)),
            pl.BlockSpec((F, HP), lambda i: (0, 0)),
        ],
        out_specs=[
            pl.BlockSpec((BN, F2), lambda i: (i, 0)),
            pl.BlockSpec((BN, F2), lambda i: (i, 0)),
            pl.BlockSpec((BN, HP), lambda i: (i, 0)),
            pl.BlockSpec((BN, HP), lambda i: (i, 0)),
            pl.BlockSpec((8, HP), lambda i: (0, 0)),
            pl.BlockSpec((8, HP), lambda i: (0, 0)),
        ],
        out_shape=[
            jax.ShapeDtypeStruct((N, F2), _f32),
            jax.ShapeDtypeStruct((N, F2), _f32),
            jax.ShapeDtypeStruct((N, HP), _f32),
            jax.ShapeDtypeStruct((N, HP), _f32),
            jax.ShapeDtypeStruct((8, HP), _f32),
            jax.ShapeDtypeStruct((8, HP), _f32),
        ],
    )(agg, es, bias, ssel, W, Asrc, Adst)


def _final_call(agg, es, bias, ssel, W_out, b_out):
    F2p = agg.shape[2]
    Fp = 2 * F2p
    K = W_out.shape[1]
    return pl.pallas_call(
        _final_body,
        grid=(N // BN,),
        in_specs=[
            pl.BlockSpec((2, BN, F2p), lambda i: (0, i, 0)),
            pl.BlockSpec((1, BN, HP), lambda i: (0, i, 0)),
            pl.BlockSpec((1, Fp), lambda i: (0, 0)),
            pl.BlockSpec((HP, Fp), lambda i: (0, 0)),
            pl.BlockSpec((Fp, K), lambda i: (0, 0)),
            pl.BlockSpec((1, K), lambda i: (0, 0)),
        ],
        out_specs=[pl.BlockSpec((BN, K), lambda i: (i, 0))],
        out_shape=[jax.ShapeDtypeStruct((N, K), _f32)],
    )(agg, es, bias, ssel, W_out, b_out)[0]


def _selw(a):
    H, C = a.shape[1], a.shape[2]
    eye = jnp.eye(HP, dtype=_f32)[:H]
    return (eye[:, None, :] * a[0][:, :, None]).reshape(H * C, HP)


def _ssel(H, C):
    return jnp.repeat(jnp.eye(HP, dtype=_f32)[:, :H], C, axis=1)


def kernel(x, edge_index, W1, a_src1, a_dst1, b1, W2, a_src2, a_dst2, b2,
           W3, a_src3, a_dst3, b3, W_out, b_out):
    src = edge_index[0].astype(jnp.int32)
    dst = edge_index[1].astype(jnp.int32)
    z16 = jnp.zeros((N, HP), _f32)
    z128 = jnp.zeros((N, 128), _f32)
    z64 = jnp.zeros((N, 64), _f32)
    z32 = jnp.zeros((N, 32), _f32)

    sc1 = _make_sc_edge_kernel(8, 32, 128)
    sc2 = _make_sc_edge_kernel(8, 16, 64)
    sc3 = _make_sc_edge_kernel(1, 64, 32)

    hlo, hhi, s, d, ms, md = _producer_call(x, W1, _selw(a_src1),
                                            _selw(a_dst1))
    agg1, es1 = sc1(s, d, hlo, hhi, src, dst, ms[0] + md[0], z128, z16)

    hlo, hhi, s, d, ms, md = _mid_call(agg1, es1, b1[None], _ssel(8, 32), W2,
                                       _selw(a_src2), _selw(a_dst2))
    agg2, es2 = sc2(s, d, hlo, hhi, src, dst, ms[0] + md[0], z64, z16)

    hlo, hhi, s, d, ms, md = _mid_call(agg2, es2, b2[None], _ssel(8, 16), W3,
                                       _selw(a_src3), _selw(a_dst3))
    agg3, es3 = sc3(s, d, hlo, hhi, src, dst, ms[0] + md[0], z32, z16)

    return _final_call(agg3, es3, b3[None], _ssel(1, 64), W_out, b_out[None])


# speedup vs baseline: 37.4076x; 2.0167x over previous
import functools

import jax
import jax.numpy as jnp
from jax import lax
from jax.experimental import pallas as pl
from jax.experimental.pallas import tpu as pltpu
from jax.experimental.pallas import tpu_sc as plsc

N = 10000
E = 320000
NC = 2
NS = 16
EPAD = 158 * 128 * NS
NP = N + 16
RPW = 624
RTAIL = NP - NS * RPW
HP = 16

_f32 = jnp.float32


def _make_sc_edge_kernel(H, C, F2, CH, NCHUNK):
    HH = max(H // 2, 1)
    VJ = (F2 // 16) // HH
    EPS = NCHUNK * CH
    assert EPS * NS >= E and EPS * NS <= EPAD and NCHUNK % 2 == 0

    mesh = plsc.VectorSubcoreMesh(core_axis_name="c", subcore_axis_name="s")

    def body(s_hbm, d_hbm, hlo_hbm, hhi_hbm, src_hbm, dst_hbm, m_hbm,
             zagg_hbm, zes_hbm,
             agg_out, es_out,
             sidx0, sidx1, didx0, didx1, sdidx0, sdidx1,
             srows0, srows1, drows0, drows1, exch0, exch1,
             hrows0, hrows1, msg0, msg1, mv,
             agg_sh, es_sh,
             isem0, isem1, gsem0, gsem1, ssem0, ssem1):
        c = lax.axis_index("c")
        s = lax.axis_index("s")
        sidx = (sidx0, sidx1)
        didx = (didx0, didx1)
        sdidx = (sdidx0, sdidx1)
        srows = (srows0, srows1)
        drows = (drows0, drows1)
        exch = (exch0, exch1)
        hrows = (hrows0, hrows1)
        msg = (msg0, msg1)
        isem = (isem0, isem1)
        gsem = (gsem0, gsem1)
        ssem = (ssem0, ssem1)

        r0 = s * RPW
        pltpu.sync_copy(zagg_hbm.at[pl.ds(r0, RPW)], agg_sh.at[pl.ds(r0, RPW)])
        pltpu.sync_copy(zes_hbm.at[pl.ds(r0, RPW)], es_sh.at[pl.ds(r0, RPW)])

        @pl.when(s == NS - 1)
        def _():
            pltpu.sync_copy(zagg_hbm.at[pl.ds(NS * RPW, RTAIL)],
                            agg_sh.at[pl.ds(NS * RPW, RTAIL)])
            pltpu.sync_copy(zes_hbm.at[pl.ds(NS * RPW, RTAIL)],
                            es_sh.at[pl.ds(NS * RPW, RTAIL)])

        pltpu.sync_copy(m_hbm, mv)
        plsc.subcore_barrier()

        ebase = s * EPS

        def issue_gathers(b):
            pltpu.async_copy(s_hbm.at[sidx[b]], srows[b], gsem[b])
            pltpu.async_copy(d_hbm.at[didx[b]], drows[b], gsem[b])

            @pl.when(c == 0)
            def _():
                pltpu.async_copy(hlo_hbm.at[sidx[b]], hrows[b], gsem[b])

            @pl.when(c == 1)
            def _():
                pltpu.async_copy(hhi_hbm.at[sidx[b]], hrows[b], gsem[b])

        def wait_gathers(b):
            pltpu.make_async_copy(s_hbm.at[sidx[b]], srows[b], gsem[b]).wait()
            pltpu.make_async_copy(d_hbm.at[didx[b]], drows[b], gsem[b]).wait()
            pltpu.make_async_copy(hlo_hbm.at[sidx[b]], hrows[b],
                                  gsem[b]).wait()

        def issue_idx(b, kk):
            base = ebase + kk * CH
            pltpu.async_copy(src_hbm.at[pl.ds(base, CH)], sidx[b], isem[b])
            pltpu.async_copy(dst_hbm.at[pl.ds(base, CH)], didx[b], isem[b])

        def wait_idx(b):
            pltpu.make_async_copy(src_hbm.at[pl.ds(0, CH)], sidx[b],
                                  isem[b]).wait()
            pltpu.make_async_copy(dst_hbm.at[pl.ds(0, CH)], didx[b],
                                  isem[b]).wait()

        def issue_scatters(b):
            pltpu.async_copy(exch[b], es_sh.at[sdidx[b]], ssem[b], add=True)
            pltpu.async_copy(msg[b], agg_sh.at[sdidx[b]], ssem[b], add=True)

        def wait_scatters(b):
            pltpu.make_async_copy(exch[b], es_sh.at[sdidx[b]],
                                  ssem[b]).wait()
            pltpu.make_async_copy(msg[b], agg_sh.at[sdidx[b]],
                                  ssem[b]).wait()

        def compute(b):
            m = mv[...]
            for i in range(CH // 16):
                sdidx[b][pl.ds(i * 16, 16)] = didx[b][pl.ds(i * 16, 16)]

            def edge_body(e, carry2):
                v = srows[b][e] + drows[b][e]
                v = jnp.where(v > 0.0, v, 0.2 * v)
                ev = jnp.exp(v - m)
                exch[b][e] = ev
                for h2 in range(HH):
                    if H > 1:
                        hd = c * HH + h2
                    else:
                        hd = h2
                    idx = jnp.full((16,), hd, dtype=jnp.int32)
                    mlt = jnp.take_along_axis(
                        ev, idx, axis=0,
                        mode=lax.GatherScatterMode.PROMISE_IN_BOUNDS)
                    for jj in range(VJ):
                        j = h2 * VJ + jj
                        msg[b][e, pl.ds(j * 16, 16)] = (
                            hrows[b][e, pl.ds(j * 16, 16)] * mlt)
                return carry2

            lax.fori_loop(0, CH, edge_body, 0)

        pltpu.sync_copy(src_hbm.at[pl.ds(ebase, CH)], sidx0)
        pltpu.sync_copy(dst_hbm.at[pl.ds(ebase, CH)], didx0)
        issue_gathers(0)
        pltpu.sync_copy(src_hbm.at[pl.ds(ebase + CH, CH)], sidx1)
        pltpu.sync_copy(dst_hbm.at[pl.ds(ebase + CH, CH)], didx1)

        def chunk_step(k, b, nb):
            @pl.when(k >= 2)
            def _():
                wait_scatters(b)

            @pl.when(k + 1 < NCHUNK)
            def _():
                @pl.when(k >= 1)
                def _():
                    wait_idx(nb)
                issue_gathers(nb)

            wait_gathers(b)

            @pl.when(k + 2 < NCHUNK)
            def _():
                issue_idx(b, k + 2)

            compute(b)
            issue_scatters(b)

        def pair_body(t, carry):
            chunk_step(2 * t, 0, 1)
            chunk_step(2 * t + 1, 1, 0)
            return carry

        lax.fori_loop(0, NCHUNK // 2, pair_body, 0)
        wait_scatters(0)
        wait_scatters(1)
        plsc.subcore_barrier()
        pltpu.sync_copy(agg_sh.at[pl.ds(r0, RPW)],
                        agg_out.at[c, pl.ds(r0, RPW)])
        pltpu.sync_copy(es_sh.at[pl.ds(r0, RPW)],
                        es_out.at[c, pl.ds(r0, RPW)])

        @pl.when(s == NS - 1)
        def _():
            pltpu.sync_copy(agg_sh.at[pl.ds(NS * RPW, RTAIL)],
                            agg_out.at[c, pl.ds(NS * RPW, RTAIL)])
            pltpu.sync_copy(es_sh.at[pl.ds(NS * RPW, RTAIL)],
                            es_out.at[c, pl.ds(NS * RPW, RTAIL)])

    return pl.kernel(
        body,
        out_type=(jax.ShapeDtypeStruct((NC, NP, F2), _f32),
                  jax.ShapeDtypeStruct((NC, NP, HP), _f32)),
        mesh=mesh,
        compiler_params=pltpu.CompilerParams(use_tc_tiling_on_sc=False),
        scratch_types=[
            pltpu.VMEM((CH,), jnp.int32),
            pltpu.VMEM((CH,), jnp.int32),
            pltpu.VMEM((CH,), jnp.int32),
            pltpu.VMEM((CH,), jnp.int32),
            pltpu.VMEM((CH,), jnp.int32),
            pltpu.VMEM((CH,), jnp.int32),
            pltpu.VMEM((CH, HP), _f32),
            pltpu.VMEM((CH, HP), _f32),
            pltpu.VMEM((CH, HP), _f32),
            pltpu.VMEM((CH, HP), _f32),
            pltpu.VMEM((CH, HP), _f32),
            pltpu.VMEM((CH, HP), _f32),
            pltpu.VMEM((CH, F2), _f32),
            pltpu.VMEM((CH, F2), _f32),
            pltpu.VMEM((CH, F2), _f32),
            pltpu.VMEM((CH, F2), _f32),
            pltpu.VMEM((16,), _f32),
            pltpu.VMEM_SHARED((NP, F2), _f32),
            pltpu.VMEM_SHARED((NP, HP), _f32),
            pltpu.SemaphoreType.DMA,
            pltpu.SemaphoreType.DMA,
            pltpu.SemaphoreType.DMA,
            pltpu.SemaphoreType.DMA,
            pltpu.SemaphoreType.DMA,
            pltpu.SemaphoreType.DMA,
        ],
    )


BN = 1000


def _dense_tail(h, asrc_ref, adst_ref, i, hlo_ref, hhi_ref, s_ref, d_ref,
                ms_ref, md_ref):
    F2 = h.shape[1] // 2
    hlo_ref[...] = h[:, :F2]
    hhi_ref[...] = h[:, F2:]
    sblk = jnp.dot(h, asrc_ref[...], preferred_element_type=_f32)
    dblk = jnp.dot(h, adst_ref[...], preferred_element_type=_f32)
    s_ref[...] = sblk
    d_ref[...] = dblk
    cms = jnp.broadcast_to(jnp.max(sblk, axis=0, keepdims=True), (8, HP))
    cmd = jnp.broadcast_to(jnp.max(dblk, axis=0, keepdims=True), (8, HP))

    @pl.when(i == 0)
    def _():
        ms_ref[...] = cms
        md_ref[...] = cmd

    @pl.when(i != 0)
    def _():
        ms_ref[...] = jnp.maximum(ms_ref[...], cms)
        md_ref[...] = jnp.maximum(md_ref[...], cmd)


def _producer_body(y_ref, w_ref, asrc_ref, adst_ref,
                   hlo_ref, hhi_ref, s_ref, d_ref, ms_ref, md_ref):
    i = pl.program_id(0)
    h = jnp.dot(y_ref[...], w_ref[...], preferred_element_type=_f32)
    _dense_tail(h, asrc_ref, adst_ref, i, hlo_ref, hhi_ref, s_ref, d_ref,
                ms_ref, md_ref)


def _elu(y):
    return jnp.where(y > 0.0, y, jnp.exp(jnp.minimum(y, 0.0)) - 1.0)


def _combine(agg_ref, es_ref, bias_ref, ssel_ref):
    ycat = jnp.concatenate([agg_ref[0], agg_ref[1]], axis=1)
    inv = 1.0 / (es_ref[0] + 1e-16)
    rep = jnp.dot(inv, ssel_ref[...], preferred_element_type=_f32)
    return _elu(ycat * rep + bias_ref[...])


def _mid_body(agg_ref, es_ref, bias_ref, ssel_ref, w_ref, asrc_ref, adst_ref,
              hlo_ref, hhi_ref, s_ref, d_ref, ms_ref, md_ref):
    i = pl.program_id(0)
    y = _combine(agg_ref, es_ref, bias_ref, ssel_ref)
    h = jnp.dot(y, w_ref[...], preferred_element_type=_f32)
    _dense_tail(h, asrc_ref, adst_ref, i, hlo_ref, hhi_ref, s_ref, d_ref,
                ms_ref, md_ref)


def _final_body(agg_ref, es_ref, bias_ref, ssel_ref, wout_ref, bout_ref,
                o_ref):
    y = _combine(agg_ref, es_ref, bias_ref, ssel_ref)
    o_ref[...] = jnp.dot(y, wout_ref[...],
                         preferred_element_type=_f32) + bout_ref[...]


def _producer_call(y, W, Asrc, Adst):
    Fin = y.shape[1]
    F = W.shape[1]
    F2 = F // 2
    return pl.pallas_call(
        _producer_body,
        grid=(N // BN,),
        in_specs=[
            pl.BlockSpec((BN, Fin), lambda i: (i, 0)),
            pl.BlockSpec((Fin, F), lambda i: (0, 0)),
            pl.BlockSpec((F, HP), lambda i: (0, 0)),
            pl.BlockSpec((F, HP), lambda i: (0, 0)),
        ],
        out_specs=[
            pl.BlockSpec((BN, F2), lambda i: (i, 0)),
            pl.BlockSpec((BN, F2), lambda i: (i, 0)),
            pl.BlockSpec((BN, HP), lambda i: (i, 0)),
            pl.BlockSpec((BN, HP), lambda i: (i, 0)),
            pl.BlockSpec((8, HP), lambda i: (0, 0)),
            pl.BlockSpec((8, HP), lambda i: (0, 0)),
        ],
        out_shape=[
            jax.ShapeDtypeStruct((N, F2), _f32),
            jax.ShapeDtypeStruct((N, F2), _f32),
            jax.ShapeDtypeStruct((N, HP), _f32),
            jax.ShapeDtypeStruct((N, HP), _f32),
            jax.ShapeDtypeStruct((8, HP), _f32),
            jax.ShapeDtypeStruct((8, HP), _f32),
        ],
    )(y, W, Asrc, Adst)


def _mid_call(agg, es, bias, ssel, W, Asrc, Adst):
    F2p = agg.shape[2]
    Fp = 2 * F2p
    F = W.shape[1]
    F2 = F // 2
    return pl.pallas_call(
        _mid_body,
        grid=(N // BN,),
        in_specs=[
            pl.BlockSpec((2, BN, F2p), lambda i: (0, i, 0)),
            pl.BlockSpec((1, BN, HP), lambda i: (0, i, 0)),
            pl.BlockSpec((1, Fp), lambda i: (0, 0)),
            pl.BlockSpec((HP, Fp), lambda i: (0, 0)),
            pl.BlockSpec((Fp, F), lambda i: (0, 0)),
            pl.BlockSpec((F, HP), lambda i: (0, 0)),
            pl.BlockSpec((F, HP), lambda i: (0, 0)),
        ],
        out_specs=[
            pl.BlockSpec((BN, F2), lambda i: (i, 0)),
            pl.BlockSpec((BN, F2), lambda i: (i, 0)),
            pl.BlockSpec((BN, HP), lambda i: (i, 0)),
            pl.BlockSpec((BN, HP), lambda i: (i, 0)),
            pl.BlockSpec((8, HP), lambda i: (0, 0)),
            pl.BlockSpec((8, HP), lambda i: (0, 0)),
        ],
        out_shape=[
            jax.ShapeDtypeStruct((N, F2), _f32),
            jax.ShapeDtypeStruct((N, F2), _f32),
            jax.ShapeDtypeStruct((N, HP), _f32),
            jax.ShapeDtypeStruct((N, HP), _f32),
            jax.ShapeDtypeStruct((8, HP), _f32),
            jax.ShapeDtypeStruct((8, HP), _f32),
        ],
    )(agg, es, bias, ssel, W, Asrc, Adst)


def _final_call(agg, es, bias, ssel, W_out, b_out):
    F2p = agg.shape[2]
    Fp = 2 * F2p
    K = W_out.shape[1]
    return pl.pallas_call(
        _final_body,
        grid=(N // BN,),
        in_specs=[
            pl.BlockSpec((2, BN, F2p), lambda i: (0, i, 0)),
            pl.BlockSpec((1, BN, HP), lambda i: (0, i, 0)),
            pl.BlockSpec((1, Fp), lambda i: (0, 0)),
            pl.BlockSpec((HP, Fp), lambda i: (0, 0)),
            pl.BlockSpec((Fp, K), lambda i: (0, 0)),
            pl.BlockSpec((1, K), lambda i: (0, 0)),
        ],
        out_specs=[pl.BlockSpec((BN, K), lambda i: (i, 0))],
        out_shape=[jax.ShapeDtypeStruct((N, K), _f32)],
    )(agg, es, bias, ssel, W_out, b_out)[0]


def _selw(a):
    H, C = a.shape[1], a.shape[2]
    eye = jnp.eye(HP, dtype=_f32)[:H]
    return (eye[:, None, :] * a[0][:, :, None]).reshape(H * C, HP)


def _ssel(H, C):
    return jnp.repeat(jnp.eye(HP, dtype=_f32)[:, :H], C, axis=1)


def _pad_nodes(a):
    return jnp.concatenate([a, jnp.zeros((NP - N, a.shape[1]), a.dtype)])


def kernel(x, edge_index, W1, a_src1, a_dst1, b1, W2, a_src2, a_dst2, b2,
           W3, a_src3, a_dst3, b3, W_out, b_out):
    src = edge_index[0].astype(jnp.int32)
    dst = edge_index[1].astype(jnp.int32)
    src = jnp.concatenate([src, jnp.zeros((EPAD - E,), jnp.int32)])
    dst = jnp.concatenate([dst, jnp.full((EPAD - E,), N, jnp.int32)])
    z16 = jnp.zeros((NP, HP), _f32)
    z128 = jnp.zeros((NP, 128), _f32)
    z64 = jnp.zeros((NP, 64), _f32)
    z32 = jnp.zeros((NP, 32), _f32)

    sc1 = _make_sc_edge_kernel(8, 32, 128, 64, 314)
    sc2 = _make_sc_edge_kernel(8, 16, 64, 128, 158)
    sc3 = _make_sc_edge_kernel(1, 64, 32, 128, 158)

    hlo, hhi, s, d, ms, md = _producer_call(x, W1, _selw(a_src1),
                                            _selw(a_dst1))
    agg1, es1 = sc1(_pad_nodes(s), _pad_nodes(d), hlo, hhi, src, dst,
                    ms[0] + md[0], z128, z16)

    hlo, hhi, s, d, ms, md = _mid_call(agg1, es1, b1[None], _ssel(8, 32), W2,
                                       _selw(a_src2), _selw(a_dst2))
    agg2, es2 = sc2(_pad_nodes(s), _pad_nodes(d), hlo, hhi, src, dst,
                    ms[0] + md[0], z64, z16)

    hlo, hhi, s, d, ms, md = _mid_call(agg2, es2, b2[None], _ssel(8, 16), W3,
                                       _selw(a_src3), _selw(a_dst3))
    agg3, es3 = sc3(_pad_nodes(s), _pad_nodes(d), hlo, hhi, src, dst,
                    ms[0] + md[0], z32, z16)

    return _final_call(agg3, es3, b3[None], _ssel(1, 64), W_out, b_out[None])


# speedup vs baseline: 94.3319x; 2.5217x over previous
import functools

import jax
import jax.numpy as jnp
from jax import lax
from jax.experimental import pallas as pl
from jax.experimental.pallas import tpu as pltpu
from jax.experimental.pallas import tpu_sc as plsc

N = 10000
E = 320000
NC = 2
NS = 16
EPAD = 158 * 128 * NS
NP = N + 16
RPW = 624
RTAIL = NP - NS * RPW
HP = 16

_f32 = jnp.float32


def _make_sc_edge_kernel(H, C, F2, CH, NCHUNK):
    HH = max(H // 2, 1)
    VJ = (F2 // 16) // HH
    EPS = NCHUNK * CH
    assert EPS * NS >= E and EPS * NS <= EPAD and NCHUNK % 2 == 0

    mesh = plsc.VectorSubcoreMesh(core_axis_name="c", subcore_axis_name="s")

    def body(s_hbm, d_hbm, hlo_hbm, hhi_hbm, src_hbm, dst_hbm, m_hbm,
             zagg_hbm, zes_hbm,
             agg_out, es_out,
             sidx0, sidx1, didx0, didx1, sdidx0, sdidx1,
             srows0, srows1, drows0, drows1, exch0, exch1,
             hrows0, hrows1, msg0, msg1, mv,
             agg_sh, es_sh,
             isem0, isem1, gsem0, gsem1, ssem0, ssem1):
        c = lax.axis_index("c")
        s = lax.axis_index("s")
        sidx = (sidx0, sidx1)
        didx = (didx0, didx1)
        sdidx = (sdidx0, sdidx1)
        srows = (srows0, srows1)
        drows = (drows0, drows1)
        exch = (exch0, exch1)
        hrows = (hrows0, hrows1)
        msg = (msg0, msg1)
        isem = (isem0, isem1)
        gsem = (gsem0, gsem1)
        ssem = (ssem0, ssem1)

        r0 = s * RPW
        pltpu.sync_copy(zagg_hbm.at[pl.ds(r0, RPW)], agg_sh.at[pl.ds(r0, RPW)])
        pltpu.sync_copy(zes_hbm.at[pl.ds(r0, RPW)], es_sh.at[pl.ds(r0, RPW)])

        @pl.when(s == NS - 1)
        def _():
            pltpu.sync_copy(zagg_hbm.at[pl.ds(NS * RPW, RTAIL)],
                            agg_sh.at[pl.ds(NS * RPW, RTAIL)])
            pltpu.sync_copy(zes_hbm.at[pl.ds(NS * RPW, RTAIL)],
                            es_sh.at[pl.ds(NS * RPW, RTAIL)])

        pltpu.sync_copy(m_hbm, mv)
        plsc.subcore_barrier()

        ebase = s * EPS

        def issue_gathers(b):
            pltpu.async_copy(s_hbm.at[sidx[b]], srows[b], gsem[b])
            pltpu.async_copy(d_hbm.at[didx[b]], drows[b], gsem[b])

            @pl.when(c == 0)
            def _():
                pltpu.async_copy(hlo_hbm.at[sidx[b]], hrows[b], gsem[b])

            @pl.when(c == 1)
            def _():
                pltpu.async_copy(hhi_hbm.at[sidx[b]], hrows[b], gsem[b])

        def wait_gathers(b):
            pltpu.make_async_copy(s_hbm.at[sidx[b]], srows[b], gsem[b]).wait()
            pltpu.make_async_copy(d_hbm.at[didx[b]], drows[b], gsem[b]).wait()
            pltpu.make_async_copy(hlo_hbm.at[sidx[b]], hrows[b],
                                  gsem[b]).wait()

        def issue_idx(b, kk):
            base = ebase + kk * CH
            pltpu.async_copy(src_hbm.at[pl.ds(base, CH)], sidx[b], isem[b])
            pltpu.async_copy(dst_hbm.at[pl.ds(base, CH)], didx[b], isem[b])

        def wait_idx(b):
            pltpu.make_async_copy(src_hbm.at[pl.ds(0, CH)], sidx[b],
                                  isem[b]).wait()
            pltpu.make_async_copy(dst_hbm.at[pl.ds(0, CH)], didx[b],
                                  isem[b]).wait()

        def issue_scatters(b):
            pltpu.async_copy(exch[b], es_sh.at[sdidx[b]], ssem[b], add=True)
            pltpu.async_copy(msg[b], agg_sh.at[sdidx[b]], ssem[b], add=True)

        def wait_scatters(b):
            pltpu.make_async_copy(exch[b], es_sh.at[sdidx[b]],
                                  ssem[b]).wait()
            pltpu.make_async_copy(msg[b], agg_sh.at[sdidx[b]],
                                  ssem[b]).wait()

        def compute(b):
            m = mv[...]
            for i in range(CH // 16):
                sdidx[b][pl.ds(i * 16, 16)] = didx[b][pl.ds(i * 16, 16)]

            @plsc.parallel_loop(0, CH, 1, unroll=8)
            def edge_body(e):
                v = srows[b][e] + drows[b][e]
                v = jnp.where(v > 0.0, v, 0.2 * v)
                ev = jnp.exp(v - m)
                exch[b][e] = ev
                for h2 in range(HH):
                    if H > 1:
                        hd = c * HH + h2
                    else:
                        hd = h2
                    idx = jnp.full((16,), hd, dtype=jnp.int32)
                    mlt = jnp.take_along_axis(
                        ev, idx, axis=0,
                        mode=lax.GatherScatterMode.PROMISE_IN_BOUNDS)
                    for jj in range(VJ):
                        j = h2 * VJ + jj
                        msg[b][e, pl.ds(j * 16, 16)] = (
                            hrows[b][e, pl.ds(j * 16, 16)] * mlt)

        pltpu.sync_copy(src_hbm.at[pl.ds(ebase, CH)], sidx0)
        pltpu.sync_copy(dst_hbm.at[pl.ds(ebase, CH)], didx0)
        issue_gathers(0)
        pltpu.sync_copy(src_hbm.at[pl.ds(ebase + CH, CH)], sidx1)
        pltpu.sync_copy(dst_hbm.at[pl.ds(ebase + CH, CH)], didx1)

        def chunk_step(k, b, nb):
            @pl.when(k >= 2)
            def _():
                wait_scatters(b)

            @pl.when(k + 1 < NCHUNK)
            def _():
                @pl.when(k >= 1)
                def _():
                    wait_idx(nb)
                issue_gathers(nb)

            wait_gathers(b)

            @pl.when(k + 2 < NCHUNK)
            def _():
                issue_idx(b, k + 2)

            compute(b)
            issue_scatters(b)

        def pair_body(t, carry):
            chunk_step(2 * t, 0, 1)
            chunk_step(2 * t + 1, 1, 0)
            return carry

        lax.fori_loop(0, NCHUNK // 2, pair_body, 0)
        wait_scatters(0)
        wait_scatters(1)
        plsc.subcore_barrier()
        pltpu.sync_copy(agg_sh.at[pl.ds(r0, RPW)],
                        agg_out.at[c, pl.ds(r0, RPW)])
        pltpu.sync_copy(es_sh.at[pl.ds(r0, RPW)],
                        es_out.at[c, pl.ds(r0, RPW)])

        @pl.when(s == NS - 1)
        def _():
            pltpu.sync_copy(agg_sh.at[pl.ds(NS * RPW, RTAIL)],
                            agg_out.at[c, pl.ds(NS * RPW, RTAIL)])
            pltpu.sync_copy(es_sh.at[pl.ds(NS * RPW, RTAIL)],
                            es_out.at[c, pl.ds(NS * RPW, RTAIL)])

    return pl.kernel(
        body,
        out_type=(jax.ShapeDtypeStruct((NC, NP, F2), _f32),
                  jax.ShapeDtypeStruct((NC, NP, HP), _f32)),
        mesh=mesh,
        compiler_params=pltpu.CompilerParams(use_tc_tiling_on_sc=False),
        scratch_types=[
            pltpu.VMEM((CH,), jnp.int32),
            pltpu.VMEM((CH,), jnp.int32),
            pltpu.VMEM((CH,), jnp.int32),
            pltpu.VMEM((CH,), jnp.int32),
            pltpu.VMEM((CH,), jnp.int32),
            pltpu.VMEM((CH,), jnp.int32),
            pltpu.VMEM((CH, HP), _f32),
            pltpu.VMEM((CH, HP), _f32),
            pltpu.VMEM((CH, HP), _f32),
            pltpu.VMEM((CH, HP), _f32),
            pltpu.VMEM((CH, HP), _f32),
            pltpu.VMEM((CH, HP), _f32),
            pltpu.VMEM((CH, F2), _f32),
            pltpu.VMEM((CH, F2), _f32),
            pltpu.VMEM((CH, F2), _f32),
            pltpu.VMEM((CH, F2), _f32),
            pltpu.VMEM((16,), _f32),
            pltpu.VMEM_SHARED((NP, F2), _f32),
            pltpu.VMEM_SHARED((NP, HP), _f32),
            pltpu.SemaphoreType.DMA,
            pltpu.SemaphoreType.DMA,
            pltpu.SemaphoreType.DMA,
            pltpu.SemaphoreType.DMA,
            pltpu.SemaphoreType.DMA,
            pltpu.SemaphoreType.DMA,
        ],
    )


BN = 1000


def _dense_tail(h, asrc_ref, adst_ref, i, hlo_ref, hhi_ref, s_ref, d_ref,
                ms_ref, md_ref):
    F2 = h.shape[1] // 2
    hlo_ref[...] = h[:, :F2]
    hhi_ref[...] = h[:, F2:]
    sblk = jnp.dot(h, asrc_ref[...], preferred_element_type=_f32)
    dblk = jnp.dot(h, adst_ref[...], preferred_element_type=_f32)
    s_ref[...] = sblk
    d_ref[...] = dblk
    cms = jnp.broadcast_to(jnp.max(sblk, axis=0, keepdims=True), (8, HP))
    cmd = jnp.broadcast_to(jnp.max(dblk, axis=0, keepdims=True), (8, HP))

    @pl.when(i == 0)
    def _():
        ms_ref[...] = cms
        md_ref[...] = cmd

    @pl.when(i != 0)
    def _():
        ms_ref[...] = jnp.maximum(ms_ref[...], cms)
        md_ref[...] = jnp.maximum(md_ref[...], cmd)


def _producer_body(y_ref, w_ref, asrc_ref, adst_ref,
                   hlo_ref, hhi_ref, s_ref, d_ref, ms_ref, md_ref):
    i = pl.program_id(0)
    h = jnp.dot(y_ref[...], w_ref[...], preferred_element_type=_f32)
    _dense_tail(h, asrc_ref, adst_ref, i, hlo_ref, hhi_ref, s_ref, d_ref,
                ms_ref, md_ref)


def _elu(y):
    return jnp.where(y > 0.0, y, jnp.exp(jnp.minimum(y, 0.0)) - 1.0)


def _combine(agg_ref, es_ref, bias_ref, ssel_ref):
    ycat = jnp.concatenate([agg_ref[0], agg_ref[1]], axis=1)
    inv = 1.0 / (es_ref[0] + 1e-16)
    rep = jnp.dot(inv, ssel_ref[...], preferred_element_type=_f32)
    return _elu(ycat * rep + bias_ref[...])


def _mid_body(agg_ref, es_ref, bias_ref, ssel_ref, w_ref, asrc_ref, adst_ref,
              hlo_ref, hhi_ref, s_ref, d_ref, ms_ref, md_ref):
    i = pl.program_id(0)
    y = _combine(agg_ref, es_ref, bias_ref, ssel_ref)
    h = jnp.dot(y, w_ref[...], preferred_element_type=_f32)
    _dense_tail(h, asrc_ref, adst_ref, i, hlo_ref, hhi_ref, s_ref, d_ref,
                ms_ref, md_ref)


def _final_body(agg_ref, es_ref, bias_ref, ssel_ref, wout_ref, bout_ref,
                o_ref):
    y = _combine(agg_ref, es_ref, bias_ref, ssel_ref)
    o_ref[...] = jnp.dot(y, wout_ref[...],
                         preferred_element_type=_f32) + bout_ref[...]


def _producer_call(y, W, Asrc, Adst):
    Fin = y.shape[1]
    F = W.shape[1]
    F2 = F // 2
    return pl.pallas_call(
        _producer_body,
        grid=(N // BN,),
        in_specs=[
            pl.BlockSpec((BN, Fin), lambda i: (i, 0)),
            pl.BlockSpec((Fin, F), lambda i: (0, 0)),
            pl.BlockSpec((F, HP), lambda i: (0, 0)),
            pl.BlockSpec((F, HP), lambda i: (0, 0)),
        ],
        out_specs=[
            pl.BlockSpec((BN, F2), lambda i: (i, 0)),
            pl.BlockSpec((BN, F2), lambda i: (i, 0)),
            pl.BlockSpec((BN, HP), lambda i: (i, 0)),
            pl.BlockSpec((BN, HP), lambda i: (i, 0)),
            pl.BlockSpec((8, HP), lambda i: (0, 0)),
            pl.BlockSpec((8, HP), lambda i: (0, 0)),
        ],
        out_shape=[
            jax.ShapeDtypeStruct((N, F2), _f32),
            jax.ShapeDtypeStruct((N, F2), _f32),
            jax.ShapeDtypeStruct((N, HP), _f32),
            jax.ShapeDtypeStruct((N, HP), _f32),
            jax.ShapeDtypeStruct((8, HP), _f32),
            jax.ShapeDtypeStruct((8, HP), _f32),
        ],
    )(y, W, Asrc, Adst)


def _mid_call(agg, es, bias, ssel, W, Asrc, Adst):
    F2p = agg.shape[2]
    Fp = 2 * F2p
    F = W.shape[1]
    F2 = F // 2
    return pl.pallas_call(
        _mid_body,
        grid=(N // BN,),
        in_specs=[
            pl.BlockSpec((2, BN, F2p), lambda i: (0, i, 0)),
            pl.BlockSpec((1, BN, HP), lambda i: (0, i, 0)),
            pl.BlockSpec((1, Fp), lambda i: (0, 0)),
            pl.BlockSpec((HP, Fp), lambda i: (0, 0)),
            pl.BlockSpec((Fp, F), lambda i: (0, 0)),
            pl.BlockSpec((F, HP), lambda i: (0, 0)),
            pl.BlockSpec((F, HP), lambda i: (0, 0)),
        ],
        out_specs=[
            pl.BlockSpec((BN, F2), lambda i: (i, 0)),
            pl.BlockSpec((BN, F2), lambda i: (i, 0)),
            pl.BlockSpec((BN, HP), lambda i: (i, 0)),
            pl.BlockSpec((BN, HP), lambda i: (i, 0)),
            pl.BlockSpec((8, HP), lambda i: (0, 0)),
            pl.BlockSpec((8, HP), lambda i: (0, 0)),
        ],
        out_shape=[
            jax.ShapeDtypeStruct((N, F2), _f32),
            jax.ShapeDtypeStruct((N, F2), _f32),
            jax.ShapeDtypeStruct((N, HP), _f32),
            jax.ShapeDtypeStruct((N, HP), _f32),
            jax.ShapeDtypeStruct((8, HP), _f32),
            jax.ShapeDtypeStruct((8, HP), _f32),
        ],
    )(agg, es, bias, ssel, W, Asrc, Adst)


def _final_call(agg, es, bias, ssel, W_out, b_out):
    F2p = agg.shape[2]
    Fp = 2 * F2p
    K = W_out.shape[1]
    return pl.pallas_call(
        _final_body,
        grid=(N // BN,),
        in_specs=[
            pl.BlockSpec((2, BN, F2p), lambda i: (0, i, 0)),
            pl.BlockSpec((1, BN, HP), lambda i: (0, i, 0)),
            pl.BlockSpec((1, Fp), lambda i: (0, 0)),
            pl.BlockSpec((HP, Fp), lambda i: (0, 0)),
            pl.BlockSpec((Fp, K), lambda i: (0, 0)),
            pl.BlockSpec((1, K), lambda i: (0, 0)),
        ],
        out_specs=[pl.BlockSpec((BN, K), lambda i: (i, 0))],
        out_shape=[jax.ShapeDtypeStruct((N, K), _f32)],
    )(agg, es, bias, ssel, W_out, b_out)[0]


def _selw(a):
    H, C = a.shape[1], a.shape[2]
    eye = jnp.eye(HP, dtype=_f32)[:H]
    return (eye[:, None, :] * a[0][:, :, None]).reshape(H * C, HP)


def _ssel(H, C):
    return jnp.repeat(jnp.eye(HP, dtype=_f32)[:, :H], C, axis=1)


def _pad_nodes(a):
    return jnp.concatenate([a, jnp.zeros((NP - N, a.shape[1]), a.dtype)])


def kernel(x, edge_index, W1, a_src1, a_dst1, b1, W2, a_src2, a_dst2, b2,
           W3, a_src3, a_dst3, b3, W_out, b_out):
    src = edge_index[0].astype(jnp.int32)
    dst = edge_index[1].astype(jnp.int32)
    src = jnp.concatenate([src, jnp.zeros((EPAD - E,), jnp.int32)])
    dst = jnp.concatenate([dst, jnp.full((EPAD - E,), N, jnp.int32)])
    z16 = jnp.zeros((NP, HP), _f32)
    z128 = jnp.zeros((NP, 128), _f32)
    z64 = jnp.zeros((NP, 64), _f32)
    z32 = jnp.zeros((NP, 32), _f32)

    sc1 = _make_sc_edge_kernel(8, 32, 128, 64, 314)
    sc2 = _make_sc_edge_kernel(8, 16, 64, 128, 158)
    sc3 = _make_sc_edge_kernel(1, 64, 32, 128, 158)

    hlo, hhi, s, d, ms, md = _producer_call(x, W1, _selw(a_src1),
                                            _selw(a_dst1))
    agg1, es1 = sc1(_pad_nodes(s), _pad_nodes(d), hlo, hhi, src, dst,
                    ms[0] + md[0], z128, z16)

    hlo, hhi, s, d, ms, md = _mid_call(agg1, es1, b1[None], _ssel(8, 32), W2,
                                       _selw(a_src2), _selw(a_dst2))
    agg2, es2 = sc2(_pad_nodes(s), _pad_nodes(d), hlo, hhi, src, dst,
                    ms[0] + md[0], z64, z16)

    hlo, hhi, s, d, ms, md = _mid_call(agg2, es2, b2[None], _ssel(8, 16), W3,
                                       _selw(a_src3), _selw(a_dst3))
    agg3, es3 = sc3(_pad_nodes(s), _pad_nodes(d), hlo, hhi, src, dst,
                    ms[0] + md[0], z32, z16)

    return _final_call(agg3, es3, b3[None], _ssel(1, 64), W_out, b_out[None])


# speedup vs baseline: 95.0865x; 1.0080x over previous
import functools

import jax
import jax.numpy as jnp
from jax import lax
from jax.experimental import pallas as pl
from jax.experimental.pallas import tpu as pltpu
from jax.experimental.pallas import tpu_sc as plsc

N = 10000
E = 320000
NC = 2
NS = 16
EPAD = 158 * 128 * NS
NP = N + 16
RPW = 624
RTAIL = NP - NS * RPW
HP = 16

_f32 = jnp.float32


def _make_sc_edge_kernel(H, C, F2, CH, NCHUNK):
    HH = max(H // 2, 1)
    VJ = (F2 // 16) // HH
    EPS = NCHUNK * CH
    assert EPS * NS >= E and EPS * NS <= EPAD and NCHUNK % 2 == 0

    mesh = plsc.VectorSubcoreMesh(core_axis_name="c", subcore_axis_name="s")

    def body(s_hbm, d_hbm, hlo_hbm, hhi_hbm, src_hbm, dst_hbm, m_hbm,
             zagg_hbm, zes_hbm,
             agg_out, es_out,
             sidx0, sidx1, didx0, didx1, sdidx0, sdidx1,
             srows0, srows1, drows0, drows1, exch0, exch1,
             hrows0, hrows1, msg0, msg1, mv,
             agg_sh, es_sh,
             isem0, isem1, gsem0, gsem1, ssem0, ssem1):
        c = lax.axis_index("c")
        s = lax.axis_index("s")
        sidx = (sidx0, sidx1)
        didx = (didx0, didx1)
        sdidx = (sdidx0, sdidx1)
        srows = (srows0, srows1)
        drows = (drows0, drows1)
        exch = (exch0, exch1)
        hrows = (hrows0, hrows1)
        msg = (msg0, msg1)
        isem = (isem0, isem1)
        gsem = (gsem0, gsem1)
        ssem = (ssem0, ssem1)

        r0 = s * RPW
        pltpu.sync_copy(zagg_hbm.at[pl.ds(r0, RPW)], agg_sh.at[pl.ds(r0, RPW)])
        pltpu.sync_copy(zes_hbm.at[pl.ds(r0, RPW)], es_sh.at[pl.ds(r0, RPW)])

        @pl.when(s == NS - 1)
        def _():
            pltpu.sync_copy(zagg_hbm.at[pl.ds(NS * RPW, RTAIL)],
                            agg_sh.at[pl.ds(NS * RPW, RTAIL)])
            pltpu.sync_copy(zes_hbm.at[pl.ds(NS * RPW, RTAIL)],
                            es_sh.at[pl.ds(NS * RPW, RTAIL)])

        pltpu.sync_copy(m_hbm, mv)
        plsc.subcore_barrier()

        ebase = s * EPS

        def issue_gathers(b):
            pltpu.async_copy(s_hbm.at[sidx[b]], srows[b], gsem[b])
            pltpu.async_copy(d_hbm.at[didx[b]], drows[b], gsem[b])

            @pl.when(c == 0)
            def _():
                pltpu.async_copy(hlo_hbm.at[sidx[b]], hrows[b], gsem[b])

            @pl.when(c == 1)
            def _():
                pltpu.async_copy(hhi_hbm.at[sidx[b]], hrows[b], gsem[b])

        def wait_gathers(b):
            pltpu.make_async_copy(s_hbm.at[sidx[b]], srows[b], gsem[b]).wait()
            pltpu.make_async_copy(d_hbm.at[didx[b]], drows[b], gsem[b]).wait()
            pltpu.make_async_copy(hlo_hbm.at[sidx[b]], hrows[b],
                                  gsem[b]).wait()

        def issue_idx(b, kk):
            base = ebase + kk * CH
            pltpu.async_copy(src_hbm.at[pl.ds(base, CH)], sidx[b], isem[b])
            pltpu.async_copy(dst_hbm.at[pl.ds(base, CH)], didx[b], isem[b])

        def wait_idx(b):
            pltpu.make_async_copy(src_hbm.at[pl.ds(0, CH)], sidx[b],
                                  isem[b]).wait()
            pltpu.make_async_copy(dst_hbm.at[pl.ds(0, CH)], didx[b],
                                  isem[b]).wait()

        def issue_scatters(b):
            pltpu.async_copy(exch[b], es_sh.at[sdidx[b]], ssem[b], add=True)
            pltpu.async_copy(msg[b], agg_sh.at[sdidx[b]], ssem[b], add=True)

        def wait_scatters(b):
            pltpu.make_async_copy(exch[b], es_sh.at[sdidx[b]],
                                  ssem[b]).wait()
            pltpu.make_async_copy(msg[b], agg_sh.at[sdidx[b]],
                                  ssem[b]).wait()

        def compute(b):
            m = mv[...]
            for i in range(CH // 16):
                sdidx[b][pl.ds(i * 16, 16)] = didx[b][pl.ds(i * 16, 16)]

            @plsc.parallel_loop(0, CH, 1, unroll=16)
            def edge_body(e):
                v = srows[b][e] + drows[b][e]
                v = jnp.where(v > 0.0, v, 0.2 * v)
                ev = jnp.exp(v - m)
                exch[b][e] = ev
                for h2 in range(HH):
                    if H > 1:
                        hd = c * HH + h2
                    else:
                        hd = h2
                    idx = jnp.full((16,), hd, dtype=jnp.int32)
                    mlt = jnp.take_along_axis(
                        ev, idx, axis=0,
                        mode=lax.GatherScatterMode.PROMISE_IN_BOUNDS)
                    for jj in range(VJ):
                        j = h2 * VJ + jj
                        msg[b][e, pl.ds(j * 16, 16)] = (
                            hrows[b][e, pl.ds(j * 16, 16)] * mlt)

        pltpu.sync_copy(src_hbm.at[pl.ds(ebase, CH)], sidx0)
        pltpu.sync_copy(dst_hbm.at[pl.ds(ebase, CH)], didx0)
        issue_gathers(0)
        pltpu.sync_copy(src_hbm.at[pl.ds(ebase + CH, CH)], sidx1)
        pltpu.sync_copy(dst_hbm.at[pl.ds(ebase + CH, CH)], didx1)

        def chunk_step(k, b, nb):
            @pl.when(k >= 2)
            def _():
                wait_scatters(b)

            @pl.when(k + 1 < NCHUNK)
            def _():
                @pl.when(k >= 1)
                def _():
                    wait_idx(nb)
                issue_gathers(nb)

            wait_gathers(b)

            @pl.when(k + 2 < NCHUNK)
            def _():
                issue_idx(b, k + 2)

            compute(b)
            issue_scatters(b)

        def pair_body(t, carry):
            chunk_step(2 * t, 0, 1)
            chunk_step(2 * t + 1, 1, 0)
            return carry

        lax.fori_loop(0, NCHUNK // 2, pair_body, 0)
        wait_scatters(0)
        wait_scatters(1)
        plsc.subcore_barrier()
        pltpu.sync_copy(agg_sh.at[pl.ds(r0, RPW)],
                        agg_out.at[c, pl.ds(r0, RPW)])
        pltpu.sync_copy(es_sh.at[pl.ds(r0, RPW)],
                        es_out.at[c, pl.ds(r0, RPW)])

        @pl.when(s == NS - 1)
        def _():
            pltpu.sync_copy(agg_sh.at[pl.ds(NS * RPW, RTAIL)],
                            agg_out.at[c, pl.ds(NS * RPW, RTAIL)])
            pltpu.sync_copy(es_sh.at[pl.ds(NS * RPW, RTAIL)],
                            es_out.at[c, pl.ds(NS * RPW, RTAIL)])

    return pl.kernel(
        body,
        out_type=(jax.ShapeDtypeStruct((NC, NP, F2), _f32),
                  jax.ShapeDtypeStruct((NC, NP, HP), _f32)),
        mesh=mesh,
        compiler_params=pltpu.CompilerParams(use_tc_tiling_on_sc=False),
        scratch_types=[
            pltpu.VMEM((CH,), jnp.int32),
            pltpu.VMEM((CH,), jnp.int32),
            pltpu.VMEM((CH,), jnp.int32),
            pltpu.VMEM((CH,), jnp.int32),
            pltpu.VMEM((CH,), jnp.int32),
            pltpu.VMEM((CH,), jnp.int32),
            pltpu.VMEM((CH, HP), _f32),
            pltpu.VMEM((CH, HP), _f32),
            pltpu.VMEM((CH, HP), _f32),
            pltpu.VMEM((CH, HP), _f32),
            pltpu.VMEM((CH, HP), _f32),
            pltpu.VMEM((CH, HP), _f32),
            pltpu.VMEM((CH, F2), _f32),
            pltpu.VMEM((CH, F2), _f32),
            pltpu.VMEM((CH, F2), _f32),
            pltpu.VMEM((CH, F2), _f32),
            pltpu.VMEM((16,), _f32),
            pltpu.VMEM_SHARED((NP, F2), _f32),
            pltpu.VMEM_SHARED((NP, HP), _f32),
            pltpu.SemaphoreType.DMA,
            pltpu.SemaphoreType.DMA,
            pltpu.SemaphoreType.DMA,
            pltpu.SemaphoreType.DMA,
            pltpu.SemaphoreType.DMA,
            pltpu.SemaphoreType.DMA,
        ],
    )


BN = 1000


def _dense_tail(h, asrc_ref, adst_ref, i, hlo_ref, hhi_ref, s_ref, d_ref,
                ms_ref, md_ref):
    F2 = h.shape[1] // 2
    hlo_ref[...] = h[:, :F2]
    hhi_ref[...] = h[:, F2:]
    sblk = jnp.dot(h, asrc_ref[...], preferred_element_type=_f32)
    dblk = jnp.dot(h, adst_ref[...], preferred_element_type=_f32)
    s_ref[...] = sblk
    d_ref[...] = dblk
    cms = jnp.broadcast_to(jnp.max(sblk, axis=0, keepdims=True), (8, HP))
    cmd = jnp.broadcast_to(jnp.max(dblk, axis=0, keepdims=True), (8, HP))

    @pl.when(i == 0)
    def _():
        ms_ref[...] = cms
        md_ref[...] = cmd

    @pl.when(i != 0)
    def _():
        ms_ref[...] = jnp.maximum(ms_ref[...], cms)
        md_ref[...] = jnp.maximum(md_ref[...], cmd)


def _producer_body(y_ref, w_ref, asrc_ref, adst_ref,
                   hlo_ref, hhi_ref, s_ref, d_ref, ms_ref, md_ref):
    i = pl.program_id(0)
    h = jnp.dot(y_ref[...], w_ref[...], preferred_element_type=_f32)
    _dense_tail(h, asrc_ref, adst_ref, i, hlo_ref, hhi_ref, s_ref, d_ref,
                ms_ref, md_ref)


def _elu(y):
    return jnp.where(y > 0.0, y, jnp.exp(jnp.minimum(y, 0.0)) - 1.0)


def _combine(agg_ref, es_ref, bias_ref, ssel_ref):
    ycat = jnp.concatenate([agg_ref[0], agg_ref[1]], axis=1)
    inv = 1.0 / (es_ref[0] + 1e-16)
    rep = jnp.dot(inv, ssel_ref[...], preferred_element_type=_f32)
    return _elu(ycat * rep + bias_ref[...])


def _mid_body(agg_ref, es_ref, bias_ref, ssel_ref, w_ref, asrc_ref, adst_ref,
              hlo_ref, hhi_ref, s_ref, d_ref, ms_ref, md_ref):
    i = pl.program_id(0)
    y = _combine(agg_ref, es_ref, bias_ref, ssel_ref)
    h = jnp.dot(y, w_ref[...], preferred_element_type=_f32)
    _dense_tail(h, asrc_ref, adst_ref, i, hlo_ref, hhi_ref, s_ref, d_ref,
                ms_ref, md_ref)


def _final_body(agg_ref, es_ref, bias_ref, ssel_ref, wout_ref, bout_ref,
                o_ref):
    y = _combine(agg_ref, es_ref, bias_ref, ssel_ref)
    o_ref[...] = jnp.dot(y, wout_ref[...],
                         preferred_element_type=_f32) + bout_ref[...]


def _producer_call(y, W, Asrc, Adst):
    Fin = y.shape[1]
    F = W.shape[1]
    F2 = F // 2
    return pl.pallas_call(
        _producer_body,
        grid=(N // BN,),
        in_specs=[
            pl.BlockSpec((BN, Fin), lambda i: (i, 0)),
            pl.BlockSpec((Fin, F), lambda i: (0, 0)),
            pl.BlockSpec((F, HP), lambda i: (0, 0)),
            pl.BlockSpec((F, HP), lambda i: (0, 0)),
        ],
        out_specs=[
            pl.BlockSpec((BN, F2), lambda i: (i, 0)),
            pl.BlockSpec((BN, F2), lambda i: (i, 0)),
            pl.BlockSpec((BN, HP), lambda i: (i, 0)),
            pl.BlockSpec((BN, HP), lambda i: (i, 0)),
            pl.BlockSpec((8, HP), lambda i: (0, 0)),
            pl.BlockSpec((8, HP), lambda i: (0, 0)),
        ],
        out_shape=[
            jax.ShapeDtypeStruct((N, F2), _f32),
            jax.ShapeDtypeStruct((N, F2), _f32),
            jax.ShapeDtypeStruct((N, HP), _f32),
            jax.ShapeDtypeStruct((N, HP), _f32),
            jax.ShapeDtypeStruct((8, HP), _f32),
            jax.ShapeDtypeStruct((8, HP), _f32),
        ],
    )(y, W, Asrc, Adst)


def _mid_call(agg, es, bias, ssel, W, Asrc, Adst):
    F2p = agg.shape[2]
    Fp = 2 * F2p
    F = W.shape[1]
    F2 = F // 2
    return pl.pallas_call(
        _mid_body,
        grid=(N // BN,),
        in_specs=[
            pl.BlockSpec((2, BN, F2p), lambda i: (0, i, 0)),
            pl.BlockSpec((1, BN, HP), lambda i: (0, i, 0)),
            pl.BlockSpec((1, Fp), lambda i: (0, 0)),
            pl.BlockSpec((HP, Fp), lambda i: (0, 0)),
            pl.BlockSpec((Fp, F), lambda i: (0, 0)),
            pl.BlockSpec((F, HP), lambda i: (0, 0)),
            pl.BlockSpec((F, HP), lambda i: (0, 0)),
        ],
        out_specs=[
            pl.BlockSpec((BN, F2), lambda i: (i, 0)),
            pl.BlockSpec((BN, F2), lambda i: (i, 0)),
            pl.BlockSpec((BN, HP), lambda i: (i, 0)),
            pl.BlockSpec((BN, HP), lambda i: (i, 0)),
            pl.BlockSpec((8, HP), lambda i: (0, 0)),
            pl.BlockSpec((8, HP), lambda i: (0, 0)),
        ],
        out_shape=[
            jax.ShapeDtypeStruct((N, F2), _f32),
            jax.ShapeDtypeStruct((N, F2), _f32),
            jax.ShapeDtypeStruct((N, HP), _f32),
            jax.ShapeDtypeStruct((N, HP), _f32),
            jax.ShapeDtypeStruct((8, HP), _f32),
            jax.ShapeDtypeStruct((8, HP), _f32),
        ],
    )(agg, es, bias, ssel, W, Asrc, Adst)


def _final_call(agg, es, bias, ssel, W_out, b_out):
    F2p = agg.shape[2]
    Fp = 2 * F2p
    K = W_out.shape[1]
    return pl.pallas_call(
        _final_body,
        grid=(N // BN,),
        in_specs=[
            pl.BlockSpec((2, BN, F2p), lambda i: (0, i, 0)),
            pl.BlockSpec((1, BN, HP), lambda i: (0, i, 0)),
            pl.BlockSpec((1, Fp), lambda i: (0, 0)),
            pl.BlockSpec((HP, Fp), lambda i: (0, 0)),
            pl.BlockSpec((Fp, K), lambda i: (0, 0)),
            pl.BlockSpec((1, K), lambda i: (0, 0)),
        ],
        out_specs=[pl.BlockSpec((BN, K), lambda i: (i, 0))],
        out_shape=[jax.ShapeDtypeStruct((N, K), _f32)],
    )(agg, es, bias, ssel, W_out, b_out)[0]


def _selw(a):
    H, C = a.shape[1], a.shape[2]
    eye = jnp.eye(HP, dtype=_f32)[:H]
    return (eye[:, None, :] * a[0][:, :, None]).reshape(H * C, HP)


def _ssel(H, C):
    return jnp.repeat(jnp.eye(HP, dtype=_f32)[:, :H], C, axis=1)


def _pad_nodes(a):
    return jnp.concatenate([a, jnp.zeros((NP - N, a.shape[1]), a.dtype)])


def kernel(x, edge_index, W1, a_src1, a_dst1, b1, W2, a_src2, a_dst2, b2,
           W3, a_src3, a_dst3, b3, W_out, b_out):
    src = edge_index[0].astype(jnp.int32)
    dst = edge_index[1].astype(jnp.int32)
    src = jnp.concatenate([src, jnp.zeros((EPAD - E,), jnp.int32)])
    dst = jnp.concatenate([dst, jnp.full((EPAD - E,), N, jnp.int32)])
    z16 = jnp.zeros((NP, HP), _f32)
    z128 = jnp.zeros((NP, 128), _f32)
    z64 = jnp.zeros((NP, 64), _f32)
    z32 = jnp.zeros((NP, 32), _f32)

    sc1 = _make_sc_edge_kernel(8, 32, 128, 64, 314)
    sc2 = _make_sc_edge_kernel(8, 16, 64, 128, 158)
    sc3 = _make_sc_edge_kernel(1, 64, 32, 128, 158)

    hlo, hhi, s, d, ms, md = _producer_call(x, W1, _selw(a_src1),
                                            _selw(a_dst1))
    agg1, es1 = sc1(_pad_nodes(s), _pad_nodes(d), hlo, hhi, src, dst,
                    ms[0] + md[0], z128, z16)

    hlo, hhi, s, d, ms, md = _mid_call(agg1, es1, b1[None], _ssel(8, 32), W2,
                                       _selw(a_src2), _selw(a_dst2))
    agg2, es2 = sc2(_pad_nodes(s), _pad_nodes(d), hlo, hhi, src, dst,
                    ms[0] + md[0], z64, z16)

    hlo, hhi, s, d, ms, md = _mid_call(agg2, es2, b2[None], _ssel(8, 16), W3,
                                       _selw(a_src3), _selw(a_dst3))
    agg3, es3 = sc3(_pad_nodes(s), _pad_nodes(d), hlo, hhi, src, dst,
                    ms[0] + md[0], z32, z16)

    return _final_call(agg3, es3, b3[None], _ssel(1, 64), W_out, b_out[None])
